# Initial kernel scaffold; baseline (speedup 1.0000x reference)
import jax
import jax.numpy as jnp
from jax import lax
from jax.experimental import pallas as pl
from jax.experimental.pallas import tpu as pltpu
from jax.experimental.pallas import tpu_sc as plsc

_N = 100000
_E = 1600000
_DIN = 128
_H = 16
_C = 40

_NC = 2
_NS = 16
_NW = _NC * _NS

_LANE = 128
_KJ = 8
_ROWS_PER_TILE = 392
_NG = _ROWS_PER_TILE // _KJ
_ROWS_TOTAL = _NW * _ROWS_PER_TILE
_EP = _ROWS_TOTAL * _LANE

_NACC = 100096
_SLICE = _NACC // _NS
_RB = _KJ * _LANE


def _seg_body(g_hbm, src_hbm, dst_hbm, out_hbm, idxs_v, idxd_v, rows_v, acc, sem):
    c = lax.axis_index("c")
    s = lax.axis_index("s")
    w = c * _NS + s
    base = w * _ROWS_PER_TILE

    zv = jnp.zeros((_H,), jnp.float32)

    def _zero_row(i, carry):
        rows_v[i, :] = zv
        return carry

    lax.fori_loop(0, _RB, _zero_row, 0)

    z0 = s * _SLICE
    for k in range(_SLICE // _RB):
        pltpu.sync_copy(rows_v, acc.at[pl.ds(z0 + k * _RB, _RB)])
    rem = _SLICE % _RB
    if rem:
        pltpu.sync_copy(rows_v.at[pl.ds(0, rem)],
                        acc.at[pl.ds(z0 + (_SLICE // _RB) * _RB, rem)])
    plsc.subcore_barrier()

    def _burst(gi, carry):
        r0 = base + gi * _KJ
        pltpu.sync_copy(src_hbm.at[pl.ds(r0, _KJ)], idxs_v)
        pltpu.sync_copy(dst_hbm.at[pl.ds(r0, _KJ)], idxd_v)
        cps = [
            pltpu.async_copy(g_hbm.at[idxs_v.at[j]],
                             rows_v.at[pl.ds(j * _LANE, _LANE)], sem)
            for j in range(_KJ)
        ]
        for cp in cps:
            cp.wait()
        for j in range(_KJ):
            pltpu.sync_copy(rows_v.at[pl.ds(j * _LANE, _LANE)],
                            acc.at[idxd_v.at[j]], add=True)
        return carry

    lax.fori_loop(0, _NG, _burst, 0)
    plsc.subcore_barrier()

    pltpu.sync_copy(acc.at[pl.ds(s * _SLICE, _SLICE)],
                    out_hbm.at[c, pl.ds(s * _SLICE, _SLICE)])


_seg_sum = pl.kernel(
    _seg_body,
    out_type=jax.ShapeDtypeStruct((_NC, _NACC, _H), jnp.float32),
    mesh=plsc.VectorSubcoreMesh(core_axis_name="c", subcore_axis_name="s"),
    scratch_types=[
        pltpu.VMEM((_KJ, _LANE), jnp.int32),
        pltpu.VMEM((_KJ, _LANE), jnp.int32),
        pltpu.VMEM((_RB, _H), jnp.float32),
        pltpu.VMEM_SHARED((_NACC, _H), jnp.float32),
        pltpu.SemaphoreType.DMA,
    ],
)


def _deg_body(dst_hbm, out_hbm, idxd_v, rows_v, acc):
    c = lax.axis_index("c")
    s = lax.axis_index("s")
    w = c * _NS + s
    base = w * _ROWS_PER_TILE

    zv = jnp.zeros((_H,), jnp.float32)

    def _zero_row(i, carry):
        rows_v[i, :] = zv
        return carry

    lax.fori_loop(0, _RB, _zero_row, 0)

    z0 = s * _SLICE
    for k in range(_SLICE // _RB):
        pltpu.sync_copy(rows_v, acc.at[pl.ds(z0 + k * _RB, _RB)])
    rem = _SLICE % _RB
    if rem:
        pltpu.sync_copy(rows_v.at[pl.ds(0, rem)],
                        acc.at[pl.ds(z0 + (_SLICE // _RB) * _RB, rem)])

    ov = jnp.ones((_H,), jnp.float32)

    def _ones_row(i, carry):
        rows_v[i, :] = ov
        return carry

    lax.fori_loop(0, _LANE, _ones_row, 0)
    plsc.subcore_barrier()

    def _burst(gi, carry):
        r0 = base + gi * _KJ
        pltpu.sync_copy(dst_hbm.at[pl.ds(r0, _KJ)], idxd_v)
        for j in range(_KJ):
            pltpu.sync_copy(rows_v.at[pl.ds(0, _LANE)],
                            acc.at[idxd_v.at[j]], add=True)
        return carry

    lax.fori_loop(0, _NG, _burst, 0)
    plsc.subcore_barrier()

    pltpu.sync_copy(acc.at[pl.ds(s * _SLICE, _SLICE)],
                    out_hbm.at[c, pl.ds(s * _SLICE, _SLICE)])


_deg_sum = pl.kernel(
    _deg_body,
    out_type=jax.ShapeDtypeStruct((_NC, _NACC, _H), jnp.float32),
    mesh=plsc.VectorSubcoreMesh(core_axis_name="c", subcore_axis_name="s"),
    scratch_types=[
        pltpu.VMEM((_KJ, _LANE), jnp.int32),
        pltpu.VMEM((_RB, _H), jnp.float32),
        pltpu.VMEM_SHARED((_NACC, _H), jnp.float32),
    ],
)


_BN = 2000
_GRID = _N // _BN


def _dinv_of(deg_ref):
    cnt = deg_ref[0, :, 0] + deg_ref[1, :, 0]
    return lax.rsqrt(cnt + 1.0)


def _tca_body(x_ref, w1_ref, deg_ref, g1_ref):
    dinv = _dinv_of(deg_ref)
    h = jnp.dot(x_ref[...], w1_ref[...], preferred_element_type=jnp.float32)
    g1_ref[...] = dinv[:, None] * h


def _tcb_body(s1_ref, g1_ref, deg_ref, b1_ref, g2_ref):
    dinv = _dinv_of(deg_ref)
    ssum = s1_ref[0] + s1_ref[1] + g1_ref[...]
    a1 = dinv[:, None] * ssum + b1_ref[...]
    g2_ref[...] = dinv[:, None] * jnp.maximum(a1, 0.0)


def _tcc_body(s2_ref, g2_ref, deg_ref, w2_ref, b2_ref, out_ref):
    dinv = _dinv_of(deg_ref)
    a2 = dinv[:, None] * (s2_ref[0] + s2_ref[1] + g2_ref[...])
    o = jnp.dot(a2, w2_ref[...], preferred_element_type=jnp.float32) + b2_ref[...]
    m = jnp.max(o, axis=1, keepdims=True)
    lse = jnp.log(jnp.sum(jnp.exp(o - m), axis=1, keepdims=True)) + m
    out_ref[...] = o - lse


def _tca(x, W1, deg2):
    return pl.pallas_call(
        _tca_body,
        grid=(_GRID,),
        in_specs=[
            pl.BlockSpec((_BN, _DIN), lambda i: (i, 0)),
            pl.BlockSpec((_DIN, _H), lambda i: (0, 0)),
            pl.BlockSpec((_NC, _BN, _H), lambda i: (0, i, 0)),
        ],
        out_specs=pl.BlockSpec((_BN, _H), lambda i: (i, 0)),
        out_shape=jax.ShapeDtypeStruct((_N, _H), jnp.float32),
    )(x, W1, deg2)


def _tcb(s1, g1, deg2, b1):
    return pl.pallas_call(
        _tcb_body,
        grid=(_GRID,),
        in_specs=[
            pl.BlockSpec((_NC, _BN, _H), lambda i: (0, i, 0)),
            pl.BlockSpec((_BN, _H), lambda i: (i, 0)),
            pl.BlockSpec((_NC, _BN, _H), lambda i: (0, i, 0)),
            pl.BlockSpec((_H,), lambda i: (0,)),
        ],
        out_specs=pl.BlockSpec((_BN, _H), lambda i: (i, 0)),
        out_shape=jax.ShapeDtypeStruct((_N, _H), jnp.float32),
    )(s1, g1, deg2, b1)


def _tcc(s2, g2, deg2, W2, b2):
    return pl.pallas_call(
        _tcc_body,
        grid=(_GRID,),
        in_specs=[
            pl.BlockSpec((_NC, _BN, _H), lambda i: (0, i, 0)),
            pl.BlockSpec((_BN, _H), lambda i: (i, 0)),
            pl.BlockSpec((_NC, _BN, _H), lambda i: (0, i, 0)),
            pl.BlockSpec((_H, _C), lambda i: (0, 0)),
            pl.BlockSpec((_C,), lambda i: (0,)),
        ],
        out_specs=pl.BlockSpec((_BN, _C), lambda i: (i, 0)),
        out_shape=jax.ShapeDtypeStruct((_N, _C), jnp.float32),
    )(s2, g2, deg2, W2, b2)


def kernel(x, edge_index, W1, b1, W2, b2):
    src = edge_index[0]
    dst = edge_index[1]
    pad = _EP - _E
    srcp = jnp.concatenate(
        [src, jnp.zeros((pad,), jnp.int32)]).reshape(_ROWS_TOTAL, _LANE)
    dstp = jnp.concatenate(
        [dst, jnp.full((pad,), _N, jnp.int32)]).reshape(_ROWS_TOTAL, _LANE)

    deg2 = _deg_sum(dstp)
    g1 = _tca(x, W1, deg2)
    s1 = _seg_sum(g1, srcp, dstp)
    g2 = _tcb(s1, g1, deg2, b1)
    s2 = _seg_sum(g2, srcp, dstp)
    return _tcc(s2, g2, deg2, W2, b2)


# speedup vs baseline: 36.0221x; 36.0221x over previous
import jax
import jax.numpy as jnp
from jax import lax
from jax.experimental import pallas as pl
from jax.experimental.pallas import tpu as pltpu
from jax.experimental.pallas import tpu_sc as plsc

_N = 100000
_E = 1600000
_DIN = 128
_H = 16
_C = 40

_NC = 2
_NS = 16
_NW = _NC * _NS

_LANE = 128
_KJ = 8
_ROWS_PER_TILE = 392
_NG = _ROWS_PER_TILE // _KJ
_ROWS_TOTAL = _NW * _ROWS_PER_TILE
_EP = _ROWS_TOTAL * _LANE

_NACC = 100096
_SLICE = _NACC // _NS
_RB = _KJ * _LANE


def _seg_body(g_hbm, src_hbm, dst_hbm, out_hbm, idxs_v, idxd_v, rows_v, acc, sem):
    c = lax.axis_index("c")
    s = lax.axis_index("s")
    w = c * _NS + s
    base = w * _ROWS_PER_TILE

    zv = jnp.zeros((_H,), jnp.float32)

    def _zero_row(i, carry):
        rows_v[i, :] = zv
        return carry

    lax.fori_loop(0, _RB, _zero_row, 0)

    z0 = s * _SLICE
    for k in range(_SLICE // _RB):
        pltpu.sync_copy(rows_v, acc.at[pl.ds(z0 + k * _RB, _RB)])
    rem = _SLICE % _RB
    if rem:
        pltpu.sync_copy(rows_v.at[pl.ds(0, rem)],
                        acc.at[pl.ds(z0 + (_SLICE // _RB) * _RB, rem)])
    plsc.subcore_barrier()

    def _burst(gi, carry):
        r0 = base + gi * _KJ
        pltpu.sync_copy(src_hbm.at[pl.ds(r0, _KJ)], idxs_v)
        pltpu.sync_copy(dst_hbm.at[pl.ds(r0, _KJ)], idxd_v)
        cps = [
            pltpu.async_copy(g_hbm.at[idxs_v.at[j]],
                             rows_v.at[pl.ds(j * _LANE, _LANE)], sem)
            for j in range(_KJ)
        ]
        for cp in cps:
            cp.wait()
        for j in range(_KJ):
            pltpu.sync_copy(rows_v.at[pl.ds(j * _LANE, _LANE)],
                            acc.at[idxd_v.at[j]], add=True)
        return carry

    lax.fori_loop(0, _NG, _burst, 0)
    plsc.subcore_barrier()

    pltpu.sync_copy(acc.at[pl.ds(s * _SLICE, _SLICE)],
                    out_hbm.at[c, pl.ds(s * _SLICE, _SLICE)])


_seg_sum = pl.kernel(
    _seg_body,
    out_type=jax.ShapeDtypeStruct((_NC, _NACC, _H), jnp.float32),
    mesh=plsc.VectorSubcoreMesh(core_axis_name="c", subcore_axis_name="s",
                                num_cores=_NC, num_subcores=_NS),
    scratch_types=[
        pltpu.VMEM((_KJ, _LANE), jnp.int32),
        pltpu.VMEM((_KJ, _LANE), jnp.int32),
        pltpu.VMEM((_RB, _H), jnp.float32),
        pltpu.VMEM_SHARED((_NACC, _H), jnp.float32),
        pltpu.SemaphoreType.DMA,
    ],
    compiler_params=pltpu.CompilerParams(use_tc_tiling_on_sc=False),
)


def _deg_body(dst_hbm, out_hbm, idxd_v, rows_v, acc):
    c = lax.axis_index("c")
    s = lax.axis_index("s")
    w = c * _NS + s
    base = w * _ROWS_PER_TILE

    zv = jnp.zeros((_H,), jnp.float32)

    def _zero_row(i, carry):
        rows_v[i, :] = zv
        return carry

    lax.fori_loop(0, _RB, _zero_row, 0)

    z0 = s * _SLICE
    for k in range(_SLICE // _RB):
        pltpu.sync_copy(rows_v, acc.at[pl.ds(z0 + k * _RB, _RB)])
    rem = _SLICE % _RB
    if rem:
        pltpu.sync_copy(rows_v.at[pl.ds(0, rem)],
                        acc.at[pl.ds(z0 + (_SLICE // _RB) * _RB, rem)])

    ov = jnp.ones((_H,), jnp.float32)

    def _ones_row(i, carry):
        rows_v[i, :] = ov
        return carry

    lax.fori_loop(0, _LANE, _ones_row, 0)
    plsc.subcore_barrier()

    def _burst(gi, carry):
        r0 = base + gi * _KJ
        pltpu.sync_copy(dst_hbm.at[pl.ds(r0, _KJ)], idxd_v)
        for j in range(_KJ):
            pltpu.sync_copy(rows_v.at[pl.ds(0, _LANE)],
                            acc.at[idxd_v.at[j]], add=True)
        return carry

    lax.fori_loop(0, _NG, _burst, 0)
    plsc.subcore_barrier()

    pltpu.sync_copy(acc.at[pl.ds(s * _SLICE, _SLICE)],
                    out_hbm.at[c, pl.ds(s * _SLICE, _SLICE)])


_deg_sum = pl.kernel(
    _deg_body,
    out_type=jax.ShapeDtypeStruct((_NC, _NACC, _H), jnp.float32),
    mesh=plsc.VectorSubcoreMesh(core_axis_name="c", subcore_axis_name="s",
                                num_cores=_NC, num_subcores=_NS),
    scratch_types=[
        pltpu.VMEM((_KJ, _LANE), jnp.int32),
        pltpu.VMEM((_RB, _H), jnp.float32),
        pltpu.VMEM_SHARED((_NACC, _H), jnp.float32),
    ],
    compiler_params=pltpu.CompilerParams(use_tc_tiling_on_sc=False),
)


_BN = 2000
_GRID = _N // _BN


def _dinv_of(deg_ref):
    cnt = deg_ref[0, :, 0] + deg_ref[1, :, 0]
    return lax.rsqrt(cnt + 1.0)


def _tca_body(x_ref, w1_ref, deg_ref, g1_ref):
    dinv = _dinv_of(deg_ref)
    h = jnp.dot(x_ref[...], w1_ref[...], preferred_element_type=jnp.float32)
    g1_ref[...] = dinv[:, None] * h


def _tcb_body(s1_ref, g1_ref, deg_ref, b1_ref, g2_ref):
    dinv = _dinv_of(deg_ref)
    ssum = s1_ref[0] + s1_ref[1] + g1_ref[...]
    a1 = dinv[:, None] * ssum + b1_ref[...]
    g2_ref[...] = dinv[:, None] * jnp.maximum(a1, 0.0)


def _tcc_body(s2_ref, g2_ref, deg_ref, w2_ref, b2_ref, out_ref):
    dinv = _dinv_of(deg_ref)
    a2 = dinv[:, None] * (s2_ref[0] + s2_ref[1] + g2_ref[...])
    o = jnp.dot(a2, w2_ref[...], preferred_element_type=jnp.float32) + b2_ref[...]
    m = jnp.max(o, axis=1, keepdims=True)
    lse = jnp.log(jnp.sum(jnp.exp(o - m), axis=1, keepdims=True)) + m
    out_ref[...] = o - lse


def _tca(x, W1, deg2):
    return pl.pallas_call(
        _tca_body,
        grid=(_GRID,),
        in_specs=[
            pl.BlockSpec((_BN, _DIN), lambda i: (i, 0)),
            pl.BlockSpec((_DIN, _H), lambda i: (0, 0)),
            pl.BlockSpec((_NC, _BN, _H), lambda i: (0, i, 0)),
        ],
        out_specs=pl.BlockSpec((_BN, _H), lambda i: (i, 0)),
        out_shape=jax.ShapeDtypeStruct((_N, _H), jnp.float32),
    )(x, W1, deg2)


def _tcb(s1, g1, deg2, b1):
    return pl.pallas_call(
        _tcb_body,
        grid=(_GRID,),
        in_specs=[
            pl.BlockSpec((_NC, _BN, _H), lambda i: (0, i, 0)),
            pl.BlockSpec((_BN, _H), lambda i: (i, 0)),
            pl.BlockSpec((_NC, _BN, _H), lambda i: (0, i, 0)),
            pl.BlockSpec((_H,), lambda i: (0,)),
        ],
        out_specs=pl.BlockSpec((_BN, _H), lambda i: (i, 0)),
        out_shape=jax.ShapeDtypeStruct((_N, _H), jnp.float32),
    )(s1, g1, deg2, b1)


def _tcc(s2, g2, deg2, W2, b2):
    return pl.pallas_call(
        _tcc_body,
        grid=(_GRID,),
        in_specs=[
            pl.BlockSpec((_NC, _BN, _H), lambda i: (0, i, 0)),
            pl.BlockSpec((_BN, _H), lambda i: (i, 0)),
            pl.BlockSpec((_NC, _BN, _H), lambda i: (0, i, 0)),
            pl.BlockSpec((_H, _C), lambda i: (0, 0)),
            pl.BlockSpec((_C,), lambda i: (0,)),
        ],
        out_specs=pl.BlockSpec((_BN, _C), lambda i: (i, 0)),
        out_shape=jax.ShapeDtypeStruct((_N, _C), jnp.float32),
    )(s2, g2, deg2, W2, b2)


def kernel(x, edge_index, W1, b1, W2, b2):
    src = edge_index[0]
    dst = edge_index[1]
    pad = _EP - _E
    srcp = jnp.concatenate(
        [src, jnp.zeros((pad,), jnp.int32)]).reshape(_ROWS_TOTAL, _LANE)
    dstp = jnp.concatenate(
        [dst, jnp.full((pad,), _N, jnp.int32)]).reshape(_ROWS_TOTAL, _LANE)

    deg2 = _deg_sum(dstp)
    g1 = _tca(x, W1, deg2)
    s1 = _seg_sum(g1, srcp, dstp)
    g2 = _tcb(s1, g1, deg2, b1)
    s2 = _seg_sum(g2, srcp, dstp)
    return _tcc(s2, g2, deg2, W2, b2)
